# baseline (speedup 1.0000x reference)
import functools

import jax
import jax.numpy as jnp
from jax import lax
from jax.experimental import pallas as pl
from jax.experimental.pallas import tpu as pltpu, tpu_sc as plsc

DIM = 64
_LANES = 16


def _make_sc_zeros():
    mesh = plsc.VectorSubcoreMesh(core_axis_name="c", subcore_axis_name="s")

    @functools.partial(
        pl.kernel,
        mesh=mesh,
        out_type=jax.ShapeDtypeStruct((1, DIM), jnp.float32),
        scratch_types=[pltpu.VMEM((DIM,), jnp.float32)],
    )
    def sc_zeros(out_hbm, buf_v):
        wid = lax.axis_index("s") * 2 + lax.axis_index("c")

        @pl.when(wid == 0)
        def _():
            for i in range(DIM // _LANES):
                buf_v[pl.ds(i * _LANES, _LANES)] = jnp.zeros(
                    (_LANES,), jnp.float32)
            pltpu.sync_copy(buf_v, out_hbm.at[0])

    return sc_zeros


_sc_zeros = _make_sc_zeros()


def kernel(inputs):
    del inputs
    return _sc_zeros()


# speedup vs baseline: 33.3373x; 33.3373x over previous
import jax
import jax.numpy as jnp
from jax.experimental import pallas as pl

DIM = 64


def _zeros_kernel(o_ref):
    o_ref[...] = jnp.zeros_like(o_ref)


def kernel(inputs):
    del inputs
    return pl.pallas_call(
        _zeros_kernel,
        out_shape=jax.ShapeDtypeStruct((1, DIM), jnp.float32),
    )()
